# speedup vs baseline: 4.3925x; 4.3925x over previous
import functools

import jax
import jax.numpy as jnp
from jax import lax
from jax.experimental import pallas as pl
from jax.experimental.pallas import tpu as pltpu
from jax.experimental.pallas import tpu_sc as plsc

N_NODES = 10000
D = 128
N_PAD = 10240
CHUNK = 128
N_SUBCORES = 16
STRIPE = N_PAD // N_SUBCORES


def _make_sc_agg(e_pad: int):
    e_per_tile = e_pad // N_SUBCORES
    n_chunks = e_per_tile // CHUNK
    mesh = plsc.VectorSubcoreMesh(core_axis_name="c", subcore_axis_name="s")

    @functools.partial(
        pl.kernel,
        out_type=[
            jax.ShapeDtypeStruct((2, N_PAD, D), jnp.float32),
            jax.ShapeDtypeStruct((2, N_PAD), jnp.float32),
        ],
        mesh=mesh,
        scratch_types=[
            pltpu.VMEM_SHARED((N_PAD, D), jnp.float32),
            pltpu.VMEM_SHARED((N_PAD,), jnp.float32),
            pltpu.VMEM((CHUNK,), jnp.int32),
            pltpu.VMEM((CHUNK,), jnp.int32),
            pltpu.VMEM((CHUNK, D), jnp.float32),
            pltpu.VMEM((CHUNK,), jnp.float32),
            pltpu.SemaphoreType.DMA,
        ],
    )
    def sc_agg(edges_hbm, x_hbm, zeros_hbm, ones_hbm,
               sums_hbm, cnts_hbm,
               acc, cacc, src_v, dst_v, rows_v, ones_v, sem):
        c = lax.axis_index("c")
        s = lax.axis_index("s")

        pltpu.sync_copy(zeros_hbm.at[pl.ds(s * STRIPE, STRIPE)],
                        acc.at[pl.ds(s * STRIPE, STRIPE)])
        for k in range(STRIPE // D):
            pltpu.sync_copy(zeros_hbm.at[0],
                            cacc.at[pl.ds(s * STRIPE + k * D, D)])
        pltpu.sync_copy(ones_hbm, ones_v)
        plsc.subcore_barrier()

        @pl.loop(0, n_chunks)
        def _chunks(j):
            base = s * e_per_tile + j * CHUNK
            pltpu.sync_copy(edges_hbm.at[c, 0, pl.ds(base, CHUNK)], src_v)
            pltpu.sync_copy(edges_hbm.at[c, 1, pl.ds(base, CHUNK)], dst_v)
            pltpu.async_copy(x_hbm.at[src_v], rows_v, sem).wait()
            pltpu.sync_copy(rows_v, acc.at[dst_v], add=True)
            pltpu.sync_copy(ones_v, cacc.at[dst_v], add=True)

        plsc.subcore_barrier()
        pltpu.sync_copy(acc.at[pl.ds(s * STRIPE, STRIPE)],
                        sums_hbm.at[c, pl.ds(s * STRIPE, STRIPE)])
        pltpu.sync_copy(cacc.at[pl.ds(s * STRIPE, STRIPE)],
                        cnts_hbm.at[c, pl.ds(s * STRIPE, STRIPE)])

    return sc_agg


def _mlp_body(x_ref, sp_ref, cp_ref, sn_ref, cn_ref,
              wp_ref, bp_ref, wn_ref, bn_ref, out_ref):
    x = x_ref[...]
    ap = sp_ref[0] / jnp.maximum(cp_ref[0], 1.0)
    an = sn_ref[0] / jnp.maximum(cn_ref[0], 1.0)
    hp = jnp.tanh(
        jnp.dot(jnp.concatenate([ap, x], axis=1), wp_ref[...],
                preferred_element_type=jnp.float32) + bp_ref[...])
    hn = jnp.tanh(
        jnp.dot(jnp.concatenate([an, x], axis=1), wn_ref[...],
                preferred_element_type=jnp.float32) + bn_ref[...])
    out_ref[...] = jnp.concatenate([hp, hn], axis=1)


def kernel(X, W_pos, b_pos, W_neg, b_neg, positive_edges, negative_edges):
    n = X.shape[0]
    e = positive_edges.shape[1]
    e_pad = -(-e // (N_SUBCORES * CHUNK)) * (N_SUBCORES * CHUNK)

    def prep(edges):
        ed = edges.astype(jnp.int32)
        pad = e_pad - e
        src = jnp.concatenate([ed[0], jnp.zeros((pad,), jnp.int32)])
        dst = jnp.concatenate([ed[1], jnp.full((pad,), n, jnp.int32)])
        return jnp.stack([src, dst])

    edges_all = jnp.stack([prep(positive_edges), prep(negative_edges)])
    zeros = jnp.zeros((N_PAD, D), jnp.float32)
    ones = jnp.ones((CHUNK,), jnp.float32)

    sums, cnts = _make_sc_agg(e_pad)(edges_all, X, zeros, ones)
    cnts = cnts.reshape(2, N_PAD, 1)

    blk = 1000
    grid = (n // blk,)
    out = pl.pallas_call(
        _mlp_body,
        grid=grid,
        in_specs=[
            pl.BlockSpec((blk, D), lambda i: (i, 0)),
            pl.BlockSpec((1, blk, D), lambda i: (0, i, 0)),
            pl.BlockSpec((1, blk, 1), lambda i: (0, i, 0)),
            pl.BlockSpec((1, blk, D), lambda i: (1, i, 0)),
            pl.BlockSpec((1, blk, 1), lambda i: (1, i, 0)),
            pl.BlockSpec((2 * D, D), lambda i: (0, 0)),
            pl.BlockSpec((1, D), lambda i: (0, 0)),
            pl.BlockSpec((2 * D, D), lambda i: (0, 0)),
            pl.BlockSpec((1, D), lambda i: (0, 0)),
        ],
        out_specs=pl.BlockSpec((blk, 2 * D), lambda i: (i, 0)),
        out_shape=jax.ShapeDtypeStruct((n, 2 * D), jnp.float32),
    )(X, sums, cnts, sums, cnts, W_pos, b_pos.reshape(1, D),
      W_neg, b_neg.reshape(1, D))
    return out
